# baseline (speedup 1.0000x reference)
import functools
import math

import jax
import jax.numpy as jnp
from jax import lax
from jax.experimental import pallas as pl
from jax.experimental.pallas import tpu as pltpu
from jax.experimental.pallas import tpu_sc as plsc

D_MODEL = 768
D_FF = 3072
N_EXP = 16
T = 2048
TILE = 128
NT = T * 2 // TILE + N_EXP
ROWS = NT * TILE
L = 16

@functools.cache
def _sc_mesh():
    return plsc.VectorSubcoreMesh(core_axis_name="c", subcore_axis_name="s")


def _router_body(x_ref, rw_ref, i1_ref, i2_ref, p1_ref, p2_ref, aux_ref):
    x = x_ref[...]
    rw = rw_ref[...]
    logits = lax.dot_general(
        x, rw, (((1,), (1,)), ((), ())), preferred_element_type=jnp.float32
    )
    m = jnp.max(logits, axis=1, keepdims=True)
    e = jnp.exp(logits - m)
    probs = e / jnp.sum(e, axis=1, keepdims=True)
    lane = lax.broadcasted_iota(jnp.int32, probs.shape, 1)
    m1 = jnp.max(probs, axis=1, keepdims=True)
    i1 = jnp.min(jnp.where(probs >= m1, lane, N_EXP), axis=1, keepdims=True)
    mask1 = lane == i1
    probs2 = jnp.where(mask1, -1.0, probs)
    m2 = jnp.max(probs2, axis=1, keepdims=True)
    i2 = jnp.min(jnp.where(probs2 >= m2, lane, N_EXP), axis=1, keepdims=True)
    denom = m1 + m2
    i1_ref[...] = i1
    i2_ref[...] = i2
    p1_ref[...] = m1 / denom
    p2_ref[...] = m2 / denom
    usage = jnp.sum(probs, axis=0, keepdims=True) * (1.0 / T)
    aux_ref[0, 0] = N_EXP * jnp.sum(usage * usage)


def _router(x_flat, router_w):
    return pl.pallas_call(
        _router_body,
        out_shape=(
            jax.ShapeDtypeStruct((T, 1), jnp.int32),
            jax.ShapeDtypeStruct((T, 1), jnp.int32),
            jax.ShapeDtypeStruct((T, 1), jnp.float32),
            jax.ShapeDtypeStruct((T, 1), jnp.float32),
            jax.ShapeDtypeStruct((1, 1), jnp.float32),
        ),
        out_specs=(
            pl.BlockSpec((T, 1), lambda: (0, 0)),
            pl.BlockSpec((T, 1), lambda: (0, 0)),
            pl.BlockSpec((T, 1), lambda: (0, 0)),
            pl.BlockSpec((T, 1), lambda: (0, 0)),
            pl.BlockSpec(memory_space=pltpu.SMEM),
        ),
    )(x_flat, router_w)


_NCHUNK = T // L


def _ds8(start, size):
    return pl.ds(pl.multiple_of(start, 8), size)


def _axis(name):
    return lax.axis_index(name)


def _scatter16(ref, pos, x, mask):
    plsc.store_scatter(ref, [pos], x, mask=mask)


def _gather_rows(src_hbm, idx_ref, dst_ref, sem):
    pltpu.async_copy(src_hbm.at[idx_ref], dst_ref, sem).wait()


def _dispatch_body(
    i1_hbm, i2_hbm, tok_hbm, te_hbm, pos1_hbm, pos2_hbm,
    i1_v, i2_v, tokloc, posc1, posc2, cvec_v, cnts_v, slab_v, mrow_v, te_v, z_v,
    counts_sh, slab1_sh, slab2_sh,
):
    core = _axis("c")
    sub = _axis("s")
    iota = jnp.arange(L, dtype=jnp.int32)

    _dispatch_scoped(
        core, sub, iota,
        i1_hbm, i2_hbm, tok_hbm, te_hbm, pos1_hbm, pos2_hbm,
        i1_v, i2_v, tokloc, posc1, posc2, cvec_v, cnts_v, slab_v, mrow_v,
        te_v, z_v, counts_sh, slab1_sh, slab2_sh,
    )


def _dispatch_scoped(
    core, sub, iota,
    i1_hbm, i2_hbm, tok_hbm, te_hbm, pos1_hbm, pos2_hbm,
    i1_v, i2_v, tokloc, posc1, posc2, cvec_v, cnts_v, slab_v, mrow_v, te_v, z_v,
    counts_sh, slab1_sh, slab2_sh,
):
    @pl.when(core == 0)
    def _():
        e = sub
        pltpu.sync_copy(i1_hbm, i1_v)
        pltpu.sync_copy(i2_hbm, i2_v)

        def count_step(c, acc):
            a = i1_v[_ds8(c * L, L)]
            b = i2_v[_ds8(c * L, L)]
            ind = jnp.logical_or(a == e, b == e)
            return acc + jnp.where(ind, 1, 0)

        acc = lax.fori_loop(0, _NCHUNK, count_step, jnp.zeros((L,), jnp.int32))
        cnt = jnp.sum(acc)
        cvec_v[...] = jnp.full((L,), cnt, jnp.int32)
        pltpu.sync_copy(cvec_v, counts_sh.at[e])
        plsc.subcore_barrier()

        pltpu.sync_copy(counts_sh, cnts_v)
        diag = jnp.zeros((L,), jnp.int32)
        for ee in range(N_EXP):
            diag = diag + jnp.where(iota == ee, cnts_v[ee, :], 0)
        padded = ((diag + (TILE - 1)) // TILE) * TILE
        csum = plsc.cumsum(padded)
        starts = csum - padded
        start_e = jnp.sum(jnp.where(iota == e, starts, 0))
        ntiles_e = jnp.sum(jnp.where(iota == e, padded, 0)) // TILE
        total_tiles = jnp.sum(jnp.where(iota == L - 1, csum, 0)) // TILE

        def zero_step(k, _):
            tokloc[_ds8(k * L, L)] = jnp.zeros((L,), jnp.int32)
            return 0

        lax.fori_loop(0, ntiles_e * (TILE // L), zero_step, 0)

        def scat_step(c, base):
            a = i1_v[_ds8(c * L, L)]
            b = i2_v[_ds8(c * L, L)]
            m1 = a == e
            m2 = b == e
            ind = jnp.logical_or(m1, m2)
            indi = jnp.where(ind, 1, 0)
            incl = plsc.cumsum(indi)
            posl = jnp.full((L,), base, jnp.int32) + incl - indi
            _scatter16(tokloc, posl, c * L + iota, ind)
            gpos = posl + jnp.full((L,), start_e, jnp.int32)
            posc1[c // 8, _ds8((c % 8) * L, L)] = jnp.where(m1, gpos, 0)
            posc2[c // 8, _ds8((c % 8) * L, L)] = jnp.where(m2, gpos, 0)
            return base + jnp.sum(indi)

        lax.fori_loop(0, _NCHUNK, scat_step, jnp.int32(0))

        def dma_step(k, _):
            pltpu.sync_copy(
                tokloc.at[_ds8(k * TILE, TILE)],
                tok_hbm.at[_ds8(start_e + k * TILE, TILE)],
            )
            return 0

        lax.fori_loop(0, ntiles_e, dma_step, 0)
        pltpu.sync_copy(posc1, slab1_sh.at[e])
        pltpu.sync_copy(posc2, slab2_sh.at[e])
        plsc.subcore_barrier()

        def merge_rows(slab, out_hbm):
            pltpu.sync_copy(slab, slab_v)
            for k in range(8):
                accv = jnp.zeros((L,), jnp.int32)
                for ee in range(N_EXP):
                    accv = accv + slab_v[ee, sub, _ds8(k * L, L)]
                mrow_v[_ds8(k * L, L)] = accv
            pltpu.sync_copy(mrow_v, out_hbm.at[sub])

        merge_rows(slab1_sh, pos1_hbm)
        merge_rows(slab2_sh, pos2_hbm)

        @pl.when(e == 0)
        def _():
            start_tiles = starts // TILE
            for j3 in range(NT // L):
                jvec = j3 * L + iota
                accv = jnp.full((L,), -1, jnp.int32)
                for ee in range(N_EXP):
                    s_ee = jnp.sum(jnp.where(iota == ee, start_tiles, 0))
                    accv = accv + jnp.where(jvec >= s_ee, 1, 0)
                te_v[_ds8(j3 * L, L)] = accv
            pltpu.sync_copy(te_v, te_hbm)

        @pl.when(e == 1)
        def _():
            for zc in range(TILE // L):
                z_v[_ds8(zc * L, L)] = jnp.zeros((L,), jnp.int32)

            def ztile(j, _):
                pltpu.sync_copy(z_v, tok_hbm.at[_ds8(j * TILE, TILE)])
                return 0

            lax.fori_loop(total_tiles, NT, ztile, 0)


def _dispatch(i1, i2):
    f = pl.kernel(
        _dispatch_body,
        out_type=(
            jax.ShapeDtypeStruct((ROWS,), jnp.int32),
            jax.ShapeDtypeStruct((NT,), jnp.int32),
            jax.ShapeDtypeStruct((L, _NCHUNK), jnp.int32),
            jax.ShapeDtypeStruct((L, _NCHUNK), jnp.int32),
        ),
        mesh=_sc_mesh(),
        compiler_params=pltpu.CompilerParams(needs_layout_passes=False),
        scratch_types=[
            pltpu.VMEM((T,), jnp.int32),
            pltpu.VMEM((T,), jnp.int32),
            pltpu.VMEM((T,), jnp.int32),
            pltpu.VMEM((L, _NCHUNK), jnp.int32),
            pltpu.VMEM((L, _NCHUNK), jnp.int32),
            pltpu.VMEM((L,), jnp.int32),
            pltpu.VMEM((L, L), jnp.int32),
            pltpu.VMEM((N_EXP, L, _NCHUNK), jnp.int32),
            pltpu.VMEM((_NCHUNK,), jnp.int32),
            pltpu.VMEM((NT,), jnp.int32),
            pltpu.VMEM((TILE,), jnp.int32),
            pltpu.VMEM_SHARED((N_EXP, L), jnp.int32),
            pltpu.VMEM_SHARED((N_EXP, L, _NCHUNK), jnp.int32),
            pltpu.VMEM_SHARED((N_EXP, L, _NCHUNK), jnp.int32),
        ],
    )
    return f(i1, i2)


_GCH = 64


def _gather_body(tok_hbm, x_hbm, xs_hbm, idx_v, rows_v, sem):
    core = _axis("c")
    sub = _axis("s")
    wid = sub * 2 + core
    per = ROWS // 32

    def step(j, _):
        base = wid * per + j * _GCH
        pltpu.sync_copy(tok_hbm.at[_ds8(base, _GCH)], idx_v)
        _gather_rows(x_hbm, idx_v, rows_v, sem)
        pltpu.sync_copy(rows_v, xs_hbm.at[_ds8(base, _GCH)])
        return 0

    lax.fori_loop(0, per // _GCH, step, 0)


def _gather(tok, x_flat):
    f = pl.kernel(
        _gather_body,
        out_type=jax.ShapeDtypeStruct((ROWS, D_MODEL), jnp.float32),
        mesh=_sc_mesh(),
        compiler_params=pltpu.CompilerParams(needs_layout_passes=False),
        scratch_types=[
            pltpu.VMEM((_GCH,), jnp.int32),
            pltpu.VMEM((_GCH, D_MODEL), jnp.float32),
            pltpu.SemaphoreType.DMA,
        ],
    )
    return f(tok, x_flat)


_FB = 1536


def _ffn_body(te_ref, xs_ref, w1_ref, w2_ref, ys_ref):
    f = pl.program_id(1)
    xb = xs_ref[...].astype(jnp.bfloat16)
    h = lax.dot_general(
        xb, w1_ref[0], (((1,), (1,)), ((), ())),
        preferred_element_type=jnp.float32,
    )
    h = 0.5 * h * (1.0 + lax.erf(h * (1.0 / math.sqrt(2.0))))
    y = lax.dot_general(
        h.astype(jnp.bfloat16), w2_ref[0], (((1,), (1,)), ((), ())),
        preferred_element_type=jnp.float32,
    )

    @pl.when(f == 0)
    def _():
        ys_ref[...] = jnp.zeros_like(ys_ref)

    ys_ref[...] += y


def _ffn(te, xs, w1, w2):
    return pl.pallas_call(
        _ffn_body,
        grid_spec=pltpu.PrefetchScalarGridSpec(
            num_scalar_prefetch=1,
            grid=(NT, D_FF // _FB),
            in_specs=[
                pl.BlockSpec((TILE, D_MODEL), lambda i, f, te_ref: (i, 0)),
                pl.BlockSpec((1, _FB, D_MODEL), lambda i, f, te_ref: (te_ref[i], f, 0)),
                pl.BlockSpec((1, D_MODEL, _FB), lambda i, f, te_ref: (te_ref[i], 0, f)),
            ],
            out_specs=pl.BlockSpec((TILE, D_MODEL), lambda i, f, te_ref: (i, 0)),
        ),
        out_shape=jax.ShapeDtypeStruct((ROWS, D_MODEL), jnp.float32),
    )(te, xs, w1, w2)


_CCH = 32


def _combine_body(
    pos1_hbm, pos2_hbm, p1_hbm, p2_hbm, ys_hbm, out_hbm,
    pv1, pv2, g1_v, g2_v, a_v, b_v, o_v, sem,
):
    core = _axis("c")
    sub = _axis("s")
    wid = sub * 2 + core
    per = T // 32

    def step(j, _):
        base = wid * per + j * _CCH
        pltpu.sync_copy(pos1_hbm.at[_ds8(base, _CCH)], pv1)
        pltpu.sync_copy(pos2_hbm.at[_ds8(base, _CCH)], pv2)
        pltpu.sync_copy(p1_hbm.at[_ds8(base, _CCH)], g1_v)
        pltpu.sync_copy(p2_hbm.at[_ds8(base, _CCH)], g2_v)
        _gather_rows(ys_hbm, pv1, a_v, sem)
        _gather_rows(ys_hbm, pv2, b_v, sem)

        def row_step(r, _):
            lm = jnp.arange(L, dtype=jnp.int32) == (r % L)
            ch = _ds8((r // L) * L, L)
            g1 = jnp.full((L,), jnp.sum(jnp.where(lm, g1_v[ch], 0.0)), jnp.float32)
            g2 = jnp.full((L,), jnp.sum(jnp.where(lm, g2_v[ch], 0.0)), jnp.float32)

            def col_step(d, _):
                o_v[r, pl.ds(d * L, L)] = (
                    g1 * a_v[r, pl.ds(d * L, L)] + g2 * b_v[r, pl.ds(d * L, L)]
                )
                return 0

            lax.fori_loop(0, D_MODEL // L, col_step, 0)
            return 0

        lax.fori_loop(0, _CCH, row_step, 0)
        pltpu.sync_copy(o_v, out_hbm.at[_ds8(base, _CCH)])
        return 0

    lax.fori_loop(0, per // _CCH, step, 0)


def _combine(pos1, pos2, p1, p2, ys):
    f = pl.kernel(
        _combine_body,
        out_type=jax.ShapeDtypeStruct((T, D_MODEL), jnp.float32),
        mesh=_sc_mesh(),
        compiler_params=pltpu.CompilerParams(needs_layout_passes=False),
        scratch_types=[
            pltpu.VMEM((_CCH,), jnp.int32),
            pltpu.VMEM((_CCH,), jnp.int32),
            pltpu.VMEM((_CCH,), jnp.float32),
            pltpu.VMEM((_CCH,), jnp.float32),
            pltpu.VMEM((_CCH, D_MODEL), jnp.float32),
            pltpu.VMEM((_CCH, D_MODEL), jnp.float32),
            pltpu.VMEM((_CCH, D_MODEL), jnp.float32),
            pltpu.SemaphoreType.DMA,
        ],
    )
    return f(pos1, pos2, p1, p2, ys)


def kernel(x, router_w, w1, w2):
    B, S, D = x.shape
    x_flat = x.reshape(-1, D)
    i1, i2, p1, p2, aux = _router(x_flat, router_w)
    i1 = i1.reshape(T)
    i2 = i2.reshape(T)
    tok, te, pos1m, pos2m = _dispatch(i1, i2)
    xs = _gather(tok, x_flat)
    ys = _ffn(te, xs, w1.astype(jnp.bfloat16), w2.astype(jnp.bfloat16))
    out = _combine(pos1m.reshape(T), pos2m.reshape(T), p1.reshape(T), p2.reshape(T), ys)
    return out.reshape(B, S, D), aux[0, 0]


# speedup vs baseline: 1.1497x; 1.1497x over previous
import functools
import math

import jax
import jax.numpy as jnp
from jax import lax
from jax.experimental import pallas as pl
from jax.experimental.pallas import tpu as pltpu
from jax.experimental.pallas import tpu_sc as plsc

D_MODEL = 768
D_FF = 3072
N_EXP = 16
T = 2048
TILE = 128
NT = T * 2 // TILE + N_EXP
ROWS = NT * TILE
L = 16

@functools.cache
def _sc_mesh():
    return plsc.VectorSubcoreMesh(core_axis_name="c", subcore_axis_name="s")


def _router_body(x_ref, rw_ref, i1_ref, i2_ref, p1_ref, p2_ref, aux_ref):
    x = x_ref[...]
    rw = rw_ref[...]
    logits = lax.dot_general(
        x, rw, (((1,), (1,)), ((), ())), preferred_element_type=jnp.float32
    )
    m = jnp.max(logits, axis=1, keepdims=True)
    e = jnp.exp(logits - m)
    probs = e / jnp.sum(e, axis=1, keepdims=True)
    lane = lax.broadcasted_iota(jnp.int32, probs.shape, 1)
    m1 = jnp.max(probs, axis=1, keepdims=True)
    i1 = jnp.min(jnp.where(probs >= m1, lane, N_EXP), axis=1, keepdims=True)
    mask1 = lane == i1
    probs2 = jnp.where(mask1, -1.0, probs)
    m2 = jnp.max(probs2, axis=1, keepdims=True)
    i2 = jnp.min(jnp.where(probs2 >= m2, lane, N_EXP), axis=1, keepdims=True)
    denom = m1 + m2
    i1_ref[...] = i1
    i2_ref[...] = i2
    p1_ref[...] = m1 / denom
    p2_ref[...] = m2 / denom
    usage = jnp.sum(probs, axis=0, keepdims=True) * (1.0 / T)
    aux_ref[0, 0] = N_EXP * jnp.sum(usage * usage)


def _router(x_flat, router_w):
    return pl.pallas_call(
        _router_body,
        out_shape=(
            jax.ShapeDtypeStruct((T, 1), jnp.int32),
            jax.ShapeDtypeStruct((T, 1), jnp.int32),
            jax.ShapeDtypeStruct((T, 1), jnp.float32),
            jax.ShapeDtypeStruct((T, 1), jnp.float32),
            jax.ShapeDtypeStruct((1, 1), jnp.float32),
        ),
        out_specs=(
            pl.BlockSpec((T, 1), lambda: (0, 0)),
            pl.BlockSpec((T, 1), lambda: (0, 0)),
            pl.BlockSpec((T, 1), lambda: (0, 0)),
            pl.BlockSpec((T, 1), lambda: (0, 0)),
            pl.BlockSpec(memory_space=pltpu.SMEM),
        ),
    )(x_flat, router_w)


_NCHUNK = T // L


def _ds8(start, size):
    return pl.ds(pl.multiple_of(start, 8), size)


def _axis(name):
    return lax.axis_index(name)


def _scatter16(ref, pos, x, mask):
    plsc.store_scatter(ref, [pos], x, mask=mask)


def _gather_rows(src_hbm, idx_ref, dst_ref, sem):
    pltpu.async_copy(src_hbm.at[idx_ref], dst_ref, sem).wait()


def _dispatch_body(
    i1_hbm, i2_hbm, tok_hbm, te_hbm, pos1_hbm, pos2_hbm,
    i1_v, i2_v, tokloc, posc1, posc2, cvec_v, cnts_v, slab_v, mrow_v, te_v, z_v,
    counts_sh, slab1_sh, slab2_sh,
):
    core = _axis("c")
    sub = _axis("s")
    iota = jnp.arange(L, dtype=jnp.int32)

    _dispatch_scoped(
        core, sub, iota,
        i1_hbm, i2_hbm, tok_hbm, te_hbm, pos1_hbm, pos2_hbm,
        i1_v, i2_v, tokloc, posc1, posc2, cvec_v, cnts_v, slab_v, mrow_v,
        te_v, z_v, counts_sh, slab1_sh, slab2_sh,
    )


def _dispatch_scoped(
    core, sub, iota,
    i1_hbm, i2_hbm, tok_hbm, te_hbm, pos1_hbm, pos2_hbm,
    i1_v, i2_v, tokloc, posc1, posc2, cvec_v, cnts_v, slab_v, mrow_v, te_v, z_v,
    counts_sh, slab1_sh, slab2_sh,
):
    @pl.when(core == 0)
    def _():
        e = sub
        pltpu.sync_copy(i1_hbm, i1_v)
        pltpu.sync_copy(i2_hbm, i2_v)

        def count_step(c, acc):
            a = i1_v[_ds8(c * L, L)]
            b = i2_v[_ds8(c * L, L)]
            ind = jnp.logical_or(a == e, b == e)
            return acc + jnp.where(ind, 1, 0)

        acc = lax.fori_loop(0, _NCHUNK, count_step, jnp.zeros((L,), jnp.int32))
        cnt = jnp.sum(acc)
        cvec_v[...] = jnp.full((L,), cnt, jnp.int32)
        pltpu.sync_copy(cvec_v, counts_sh.at[e])
        plsc.subcore_barrier()

        pltpu.sync_copy(counts_sh, cnts_v)
        diag = jnp.zeros((L,), jnp.int32)
        for ee in range(N_EXP):
            diag = diag + jnp.where(iota == ee, cnts_v[ee, :], 0)
        padded = ((diag + (TILE - 1)) // TILE) * TILE
        csum = plsc.cumsum(padded)
        starts = csum - padded
        start_e = jnp.sum(jnp.where(iota == e, starts, 0))
        ntiles_e = jnp.sum(jnp.where(iota == e, padded, 0)) // TILE
        total_tiles = jnp.sum(jnp.where(iota == L - 1, csum, 0)) // TILE

        def zero_step(k, _):
            tokloc[_ds8(k * L, L)] = jnp.zeros((L,), jnp.int32)
            return 0

        lax.fori_loop(0, ntiles_e * (TILE // L), zero_step, 0)

        def scat_step(c, base):
            a = i1_v[_ds8(c * L, L)]
            b = i2_v[_ds8(c * L, L)]
            m1 = a == e
            m2 = b == e
            ind = jnp.logical_or(m1, m2)
            indi = jnp.where(ind, 1, 0)
            incl = plsc.cumsum(indi)
            posl = jnp.full((L,), base, jnp.int32) + incl - indi
            _scatter16(tokloc, posl, c * L + iota, ind)
            gpos = posl + jnp.full((L,), start_e, jnp.int32)
            posc1[c // 8, _ds8((c % 8) * L, L)] = jnp.where(m1, gpos, 0)
            posc2[c // 8, _ds8((c % 8) * L, L)] = jnp.where(m2, gpos, 0)
            return base + jnp.sum(indi)

        lax.fori_loop(0, _NCHUNK, scat_step, jnp.int32(0))

        def dma_step(k, _):
            pltpu.sync_copy(
                tokloc.at[_ds8(k * TILE, TILE)],
                tok_hbm.at[_ds8(start_e + k * TILE, TILE)],
            )
            return 0

        lax.fori_loop(0, ntiles_e, dma_step, 0)
        pltpu.sync_copy(posc1, slab1_sh.at[e])
        pltpu.sync_copy(posc2, slab2_sh.at[e])
        plsc.subcore_barrier()

        def merge_rows(slab, out_hbm):
            pltpu.sync_copy(slab, slab_v)
            for k in range(8):
                accv = jnp.zeros((L,), jnp.int32)
                for ee in range(N_EXP):
                    accv = accv + slab_v[ee, sub, _ds8(k * L, L)]
                mrow_v[_ds8(k * L, L)] = accv
            pltpu.sync_copy(mrow_v, out_hbm.at[sub])

        merge_rows(slab1_sh, pos1_hbm)
        merge_rows(slab2_sh, pos2_hbm)

        @pl.when(e == 0)
        def _():
            start_tiles = starts // TILE
            for j3 in range(NT // L):
                jvec = j3 * L + iota
                accv = jnp.full((L,), -1, jnp.int32)
                for ee in range(N_EXP):
                    s_ee = jnp.sum(jnp.where(iota == ee, start_tiles, 0))
                    accv = accv + jnp.where(jvec >= s_ee, 1, 0)
                te_v[_ds8(j3 * L, L)] = accv
            te_v[_ds8(NT, L)] = jnp.where(
                iota == 0, jnp.full((L,), total_tiles, jnp.int32), 0
            )
            pltpu.sync_copy(te_v, te_hbm)

        @pl.when(e == 1)
        def _():
            for zc in range(TILE // L):
                z_v[_ds8(zc * L, L)] = jnp.zeros((L,), jnp.int32)

            def ztile(j, _):
                pltpu.sync_copy(z_v, tok_hbm.at[_ds8(j * TILE, TILE)])
                return 0

            lax.fori_loop(total_tiles, NT, ztile, 0)


def _dispatch(i1, i2):
    f = pl.kernel(
        _dispatch_body,
        out_type=(
            jax.ShapeDtypeStruct((ROWS,), jnp.int32),
            jax.ShapeDtypeStruct((NT + L,), jnp.int32),
            jax.ShapeDtypeStruct((L, _NCHUNK), jnp.int32),
            jax.ShapeDtypeStruct((L, _NCHUNK), jnp.int32),
        ),
        mesh=_sc_mesh(),
        compiler_params=pltpu.CompilerParams(needs_layout_passes=False),
        scratch_types=[
            pltpu.VMEM((T,), jnp.int32),
            pltpu.VMEM((T,), jnp.int32),
            pltpu.VMEM((T,), jnp.int32),
            pltpu.VMEM((L, _NCHUNK), jnp.int32),
            pltpu.VMEM((L, _NCHUNK), jnp.int32),
            pltpu.VMEM((L,), jnp.int32),
            pltpu.VMEM((L, L), jnp.int32),
            pltpu.VMEM((N_EXP, L, _NCHUNK), jnp.int32),
            pltpu.VMEM((_NCHUNK,), jnp.int32),
            pltpu.VMEM((NT + L,), jnp.int32),
            pltpu.VMEM((TILE,), jnp.int32),
            pltpu.VMEM_SHARED((N_EXP, L), jnp.int32),
            pltpu.VMEM_SHARED((N_EXP, L, _NCHUNK), jnp.int32),
            pltpu.VMEM_SHARED((N_EXP, L, _NCHUNK), jnp.int32),
        ],
    )
    return f(i1, i2)


_GCH = 48
_GNB = 3


def _gather_body(tok_hbm, x_hbm, xs_hbm, idx_v, r0, r1, r2, g0, g1, g2, w0, w1s, w2s):
    core = _axis("c")
    sub = _axis("s")
    wid = sub * 2 + core
    per = ROWS // 32
    base = wid * per
    bufs = (r0, r1, r2)
    gsems = (g0, g1, g2)
    wsems = (w0, w1s, w2s)
    nch = per // _GCH

    pltpu.sync_copy(tok_hbm.at[_ds8(base, per)], idx_v)
    gd = [None] * nch
    wd = [None] * nch
    for j in range(nch):
        b = j % _GNB
        if j >= _GNB:
            wd[j - _GNB].wait()
        gd[j] = pltpu.make_async_copy(
            x_hbm.at[idx_v.at[_ds8(j * _GCH, _GCH)]], bufs[b], gsems[b]
        )
        gd[j].start()
        if j >= 1:
            p = j - 1
            gd[p].wait()
            wd[p] = pltpu.make_async_copy(
                bufs[p % _GNB], xs_hbm.at[_ds8(base + p * _GCH, _GCH)], wsems[p % _GNB]
            )
            wd[p].start()
    gd[nch - 1].wait()
    wd[nch - 1] = pltpu.make_async_copy(
        bufs[(nch - 1) % _GNB],
        xs_hbm.at[_ds8(base + (nch - 1) * _GCH, _GCH)],
        wsems[(nch - 1) % _GNB],
    )
    wd[nch - 1].start()
    for j in range(max(0, nch - _GNB), nch):
        wd[j].wait()


def _gather(tok, x_flat):
    f = pl.kernel(
        _gather_body,
        out_type=jax.ShapeDtypeStruct((ROWS, D_MODEL), jnp.float32),
        mesh=_sc_mesh(),
        compiler_params=pltpu.CompilerParams(needs_layout_passes=False),
        scratch_types=[
            pltpu.VMEM((ROWS // 32,), jnp.int32),
            pltpu.VMEM((_GCH, D_MODEL), jnp.float32),
            pltpu.VMEM((_GCH, D_MODEL), jnp.float32),
            pltpu.VMEM((_GCH, D_MODEL), jnp.float32),
            pltpu.SemaphoreType.DMA,
            pltpu.SemaphoreType.DMA,
            pltpu.SemaphoreType.DMA,
            pltpu.SemaphoreType.DMA,
            pltpu.SemaphoreType.DMA,
            pltpu.SemaphoreType.DMA,
        ],
    )
    return f(tok, x_flat)


_FB = 1536


def _ffn_body(te_ref, xs_ref, w1_ref, w2_ref, ys_ref):
    i = pl.program_id(0)
    f = pl.program_id(1)
    nact = te_ref[NT]

    @pl.when(i < nact)
    def _():
        _ffn_tile(f, xs_ref, w1_ref, w2_ref, ys_ref)


def _ffn_tile(f, xs_ref, w1_ref, w2_ref, ys_ref):
    h = lax.dot_general(
        xs_ref[...], w1_ref[0], (((1,), (1,)), ((), ())),
        preferred_element_type=jnp.float32,
    )
    h = 0.5 * h * (1.0 + lax.erf(h * (1.0 / math.sqrt(2.0))))
    y = lax.dot_general(
        h, w2_ref[0], (((1,), (1,)), ((), ())),
        preferred_element_type=jnp.float32,
    )

    @pl.when(f == 0)
    def _():
        ys_ref[...] = jnp.zeros_like(ys_ref)

    ys_ref[...] += y


def _ffn(te, xs, w1, w2):
    return pl.pallas_call(
        _ffn_body,
        grid_spec=pltpu.PrefetchScalarGridSpec(
            num_scalar_prefetch=1,
            grid=(NT, D_FF // _FB),
            in_specs=[
                pl.BlockSpec((TILE, D_MODEL), lambda i, f, te_ref: (i, 0)),
                pl.BlockSpec((1, _FB, D_MODEL), lambda i, f, te_ref: (te_ref[i], f, 0)),
                pl.BlockSpec((1, D_MODEL, _FB), lambda i, f, te_ref: (te_ref[i], 0, f)),
            ],
            out_specs=pl.BlockSpec((TILE, D_MODEL), lambda i, f, te_ref: (i, 0)),
        ),
        out_shape=jax.ShapeDtypeStruct((ROWS, D_MODEL), jnp.float32),
    )(te, xs, w1, w2)


_CCH = 32


def _combine_body(
    pos1_hbm, pos2_hbm, p1_hbm, p2_hbm, ys_hbm, out_hbm,
    pv1, pv2, g1_v, g2_v, a_v, b_v, o_v, sem,
):
    core = _axis("c")
    sub = _axis("s")
    wid = sub * 2 + core
    per = T // 32

    def step(j, _):
        base = wid * per + j * _CCH
        pltpu.sync_copy(pos1_hbm.at[_ds8(base, _CCH)], pv1)
        pltpu.sync_copy(pos2_hbm.at[_ds8(base, _CCH)], pv2)
        pltpu.sync_copy(p1_hbm.at[_ds8(base, _CCH)], g1_v)
        pltpu.sync_copy(p2_hbm.at[_ds8(base, _CCH)], g2_v)
        _gather_rows(ys_hbm, pv1, a_v, sem)
        _gather_rows(ys_hbm, pv2, b_v, sem)

        def row_step(r, _):
            lm = jnp.arange(L, dtype=jnp.int32) == (r % L)
            ch = _ds8((r // L) * L, L)
            g1 = jnp.full((L,), jnp.sum(jnp.where(lm, g1_v[ch], 0.0)), jnp.float32)
            g2 = jnp.full((L,), jnp.sum(jnp.where(lm, g2_v[ch], 0.0)), jnp.float32)

            def col_step(d, _):
                o_v[r, pl.ds(d * L, L)] = (
                    g1 * a_v[r, pl.ds(d * L, L)] + g2 * b_v[r, pl.ds(d * L, L)]
                )
                return 0

            lax.fori_loop(0, D_MODEL // L, col_step, 0)
            return 0

        lax.fori_loop(0, _CCH, row_step, 0)
        pltpu.sync_copy(o_v, out_hbm.at[_ds8(base, _CCH)])
        return 0

    lax.fori_loop(0, per // _CCH, step, 0)


def _combine(pos1, pos2, p1, p2, ys):
    f = pl.kernel(
        _combine_body,
        out_type=jax.ShapeDtypeStruct((T, D_MODEL), jnp.float32),
        mesh=_sc_mesh(),
        compiler_params=pltpu.CompilerParams(needs_layout_passes=False),
        scratch_types=[
            pltpu.VMEM((_CCH,), jnp.int32),
            pltpu.VMEM((_CCH,), jnp.int32),
            pltpu.VMEM((_CCH,), jnp.float32),
            pltpu.VMEM((_CCH,), jnp.float32),
            pltpu.VMEM((_CCH, D_MODEL), jnp.float32),
            pltpu.VMEM((_CCH, D_MODEL), jnp.float32),
            pltpu.VMEM((_CCH, D_MODEL), jnp.float32),
            pltpu.SemaphoreType.DMA,
        ],
    )
    return f(pos1, pos2, p1, p2, ys)


def kernel(x, router_w, w1, w2):
    B, S, D = x.shape
    x_flat = x.reshape(-1, D)
    i1, i2, p1, p2, aux = _router(x_flat, router_w)
    i1 = i1.reshape(T)
    i2 = i2.reshape(T)
    tok, te, pos1m, pos2m = _dispatch(i1, i2)
    xs = _gather(tok, x_flat)
    ys = _ffn(te, xs, w1, w2)
    out = _combine(pos1m.reshape(T), pos2m.reshape(T), p1.reshape(T), p2.reshape(T), ys)
    return out.reshape(B, S, D), aux[0, 0]


# speedup vs baseline: 1.4383x; 1.2510x over previous
import functools
import math

import jax
import jax.numpy as jnp
from jax import lax
from jax.experimental import pallas as pl
from jax.experimental.pallas import tpu as pltpu
from jax.experimental.pallas import tpu_sc as plsc

D_MODEL = 768
D_FF = 3072
N_EXP = 16
T = 2048
TILE = 128
NT = T * 2 // TILE + N_EXP
ROWS = NT * TILE
L = 16

@functools.cache
def _sc_mesh():
    return plsc.VectorSubcoreMesh(core_axis_name="c", subcore_axis_name="s")


def _router_body(x_ref, rw_ref, i1_ref, i2_ref, p1_ref, p2_ref, aux_ref):
    x = x_ref[...]
    rw = rw_ref[...]
    logits = lax.dot_general(
        x, rw, (((1,), (1,)), ((), ())), preferred_element_type=jnp.float32
    )
    m = jnp.max(logits, axis=1, keepdims=True)
    e = jnp.exp(logits - m)
    probs = e / jnp.sum(e, axis=1, keepdims=True)
    lane = lax.broadcasted_iota(jnp.int32, probs.shape, 1)
    m1 = jnp.max(probs, axis=1, keepdims=True)
    i1 = jnp.min(jnp.where(probs >= m1, lane, N_EXP), axis=1, keepdims=True)
    mask1 = lane == i1
    probs2 = jnp.where(mask1, -1.0, probs)
    m2 = jnp.max(probs2, axis=1, keepdims=True)
    i2 = jnp.min(jnp.where(probs2 >= m2, lane, N_EXP), axis=1, keepdims=True)
    denom = m1 + m2
    i1_ref[...] = i1
    i2_ref[...] = i2
    p1_ref[...] = m1 / denom
    p2_ref[...] = m2 / denom
    usage = jnp.sum(probs, axis=0, keepdims=True) * (1.0 / T)
    aux_ref[0, 0] = N_EXP * jnp.sum(usage * usage)


def _router(x_flat, router_w):
    return pl.pallas_call(
        _router_body,
        out_shape=(
            jax.ShapeDtypeStruct((T, 1), jnp.int32),
            jax.ShapeDtypeStruct((T, 1), jnp.int32),
            jax.ShapeDtypeStruct((T, 1), jnp.float32),
            jax.ShapeDtypeStruct((T, 1), jnp.float32),
            jax.ShapeDtypeStruct((1, 1), jnp.float32),
        ),
        out_specs=(
            pl.BlockSpec((T, 1), lambda: (0, 0)),
            pl.BlockSpec((T, 1), lambda: (0, 0)),
            pl.BlockSpec((T, 1), lambda: (0, 0)),
            pl.BlockSpec((T, 1), lambda: (0, 0)),
            pl.BlockSpec(memory_space=pltpu.SMEM),
        ),
    )(x_flat, router_w)


_NCHUNK = T // L


def _ds8(start, size):
    return pl.ds(pl.multiple_of(start, 8), size)


def _axis(name):
    return lax.axis_index(name)


def _scatter16(ref, pos, x, mask):
    plsc.store_scatter(ref, [pos], x, mask=mask)


def _gather_rows(src_hbm, idx_ref, dst_ref, sem):
    pltpu.async_copy(src_hbm.at[idx_ref], dst_ref, sem).wait()


def _dispatch_body(
    i1_hbm, i2_hbm, tok_hbm, te_hbm, pos1_hbm, pos2_hbm,
    i1_v, i2_v, tokloc, posc1, posc2, cvec_v, cnts_v, slab_v, mrow_v, te_v, z_v,
    counts_sh, slab1_sh, slab2_sh,
):
    core = _axis("c")
    sub = _axis("s")
    iota = jnp.arange(L, dtype=jnp.int32)

    _dispatch_scoped(
        core, sub, iota,
        i1_hbm, i2_hbm, tok_hbm, te_hbm, pos1_hbm, pos2_hbm,
        i1_v, i2_v, tokloc, posc1, posc2, cvec_v, cnts_v, slab_v, mrow_v,
        te_v, z_v, counts_sh, slab1_sh, slab2_sh,
    )


def _dispatch_scoped(
    core, sub, iota,
    i1_hbm, i2_hbm, tok_hbm, te_hbm, pos1_hbm, pos2_hbm,
    i1_v, i2_v, tokloc, posc1, posc2, cvec_v, cnts_v, slab_v, mrow_v, te_v, z_v,
    counts_sh, slab1_sh, slab2_sh,
):
    @pl.when(core == 0)
    def _():
        e = sub
        pltpu.sync_copy(i1_hbm, i1_v)
        pltpu.sync_copy(i2_hbm, i2_v)

        def count_step(c, acc):
            a = i1_v[_ds8(c * L, L)]
            b = i2_v[_ds8(c * L, L)]
            ind = jnp.logical_or(a == e, b == e)
            return acc + jnp.where(ind, 1, 0)

        acc = lax.fori_loop(0, _NCHUNK, count_step, jnp.zeros((L,), jnp.int32))
        cnt = jnp.sum(acc)
        cvec_v[...] = jnp.full((L,), cnt, jnp.int32)
        pltpu.sync_copy(cvec_v, counts_sh.at[e])
        plsc.subcore_barrier()

        pltpu.sync_copy(counts_sh, cnts_v)
        diag = jnp.zeros((L,), jnp.int32)
        for ee in range(N_EXP):
            diag = diag + jnp.where(iota == ee, cnts_v[ee, :], 0)
        padded = ((diag + (TILE - 1)) // TILE) * TILE
        csum = plsc.cumsum(padded)
        starts = csum - padded
        start_e = jnp.sum(jnp.where(iota == e, starts, 0))
        ntiles_e = jnp.sum(jnp.where(iota == e, padded, 0)) // TILE
        total_tiles = jnp.sum(jnp.where(iota == L - 1, csum, 0)) // TILE

        def zero_step(k, _):
            tokloc[_ds8(k * L, L)] = jnp.zeros((L,), jnp.int32)
            return 0

        lax.fori_loop(0, ntiles_e * (TILE // L), zero_step, 0)

        def scat_step(c, base):
            a = i1_v[_ds8(c * L, L)]
            b = i2_v[_ds8(c * L, L)]
            m1 = a == e
            m2 = b == e
            ind = jnp.logical_or(m1, m2)
            indi = jnp.where(ind, 1, 0)
            incl = plsc.cumsum(indi)
            posl = jnp.full((L,), base, jnp.int32) + incl - indi
            _scatter16(tokloc, posl, c * L + iota, ind)
            gpos = posl + jnp.full((L,), start_e, jnp.int32)
            posc1[c // 8, _ds8((c % 8) * L, L)] = jnp.where(m1, gpos, 0)
            posc2[c // 8, _ds8((c % 8) * L, L)] = jnp.where(m2, gpos, 0)
            return base + jnp.sum(indi)

        lax.fori_loop(0, _NCHUNK, scat_step, jnp.int32(0))

        def dma_step(k, _):
            pltpu.sync_copy(
                tokloc.at[_ds8(k * TILE, TILE)],
                tok_hbm.at[_ds8(start_e + k * TILE, TILE)],
            )
            return 0

        lax.fori_loop(0, ntiles_e, dma_step, 0)
        pltpu.sync_copy(posc1, slab1_sh.at[e])
        pltpu.sync_copy(posc2, slab2_sh.at[e])
        plsc.subcore_barrier()

        def merge_rows(slab, out_hbm):
            pltpu.sync_copy(slab, slab_v)
            for k in range(8):
                accv = jnp.zeros((L,), jnp.int32)
                for ee in range(N_EXP):
                    accv = accv + slab_v[ee, sub, _ds8(k * L, L)]
                mrow_v[_ds8(k * L, L)] = accv
            pltpu.sync_copy(mrow_v, out_hbm.at[sub])

        merge_rows(slab1_sh, pos1_hbm)
        merge_rows(slab2_sh, pos2_hbm)

        @pl.when(e == 0)
        def _():
            start_tiles = starts // TILE
            for j3 in range(NT // L):
                jvec = j3 * L + iota
                accv = jnp.full((L,), -1, jnp.int32)
                for ee in range(N_EXP):
                    s_ee = jnp.sum(jnp.where(iota == ee, start_tiles, 0))
                    accv = accv + jnp.where(jvec >= s_ee, 1, 0)
                te_v[_ds8(j3 * L, L)] = accv
            te_v[_ds8(NT, L)] = jnp.where(
                iota == 0, jnp.full((L,), total_tiles, jnp.int32), 0
            )
            pltpu.sync_copy(te_v, te_hbm)

        @pl.when(e == 1)
        def _():
            for zc in range(TILE // L):
                z_v[_ds8(zc * L, L)] = jnp.zeros((L,), jnp.int32)

            def ztile(j, _):
                pltpu.sync_copy(z_v, tok_hbm.at[_ds8(j * TILE, TILE)])
                return 0

            lax.fori_loop(total_tiles, NT, ztile, 0)


def _dispatch(i1, i2):
    f = pl.kernel(
        _dispatch_body,
        out_type=(
            jax.ShapeDtypeStruct((ROWS,), jnp.int32),
            jax.ShapeDtypeStruct((NT + L,), jnp.int32),
            jax.ShapeDtypeStruct((L, _NCHUNK), jnp.int32),
            jax.ShapeDtypeStruct((L, _NCHUNK), jnp.int32),
        ),
        mesh=_sc_mesh(),
        compiler_params=pltpu.CompilerParams(needs_layout_passes=False),
        scratch_types=[
            pltpu.VMEM((T,), jnp.int32),
            pltpu.VMEM((T,), jnp.int32),
            pltpu.VMEM((T,), jnp.int32),
            pltpu.VMEM((L, _NCHUNK), jnp.int32),
            pltpu.VMEM((L, _NCHUNK), jnp.int32),
            pltpu.VMEM((L,), jnp.int32),
            pltpu.VMEM((L, L), jnp.int32),
            pltpu.VMEM((N_EXP, L, _NCHUNK), jnp.int32),
            pltpu.VMEM((_NCHUNK,), jnp.int32),
            pltpu.VMEM((NT + L,), jnp.int32),
            pltpu.VMEM((TILE,), jnp.int32),
            pltpu.VMEM_SHARED((N_EXP, L), jnp.int32),
            pltpu.VMEM_SHARED((N_EXP, L, _NCHUNK), jnp.int32),
            pltpu.VMEM_SHARED((N_EXP, L, _NCHUNK), jnp.int32),
        ],
    )
    return f(i1, i2)


_FB = 1536


def _ffn_body(te_ref, tok_ref, x_ref, w1_ref, w2_ref, ys_ref, xg_ref):
    i = pl.program_id(0)
    f = pl.program_id(1)
    nact = te_ref[NT]

    @pl.when(i < nact)
    def _():
        @pl.when(f == 0)
        def _():
            tok_b = tok_ref[0]
            iota_t = lax.broadcasted_iota(jnp.int32, (T, TILE), 0)
            oh = jnp.where(iota_t == tok_b, 1.0, 0.0)
            xg_ref[...] = lax.dot_general(
                oh, x_ref[...], (((0,), (0,)), ((), ())),
                preferred_element_type=jnp.float32,
            )

        h = lax.dot_general(
            xg_ref[...], w1_ref[0], (((1,), (1,)), ((), ())),
            preferred_element_type=jnp.float32,
        )
        h = 0.5 * h * (1.0 + lax.erf(h * (1.0 / math.sqrt(2.0))))
        y = lax.dot_general(
            h, w2_ref[0], (((1,), (1,)), ((), ())),
            preferred_element_type=jnp.float32,
        )

        @pl.when(f == 0)
        def _():
            ys_ref[...] = jnp.zeros_like(ys_ref)

        ys_ref[...] += y


def _ffn(te, tok, x_flat, w1, w2):
    return pl.pallas_call(
        _ffn_body,
        grid_spec=pltpu.PrefetchScalarGridSpec(
            num_scalar_prefetch=1,
            grid=(NT, D_FF // _FB),
            in_specs=[
                pl.BlockSpec((1, 1, TILE), lambda i, f, te_ref: (i, 0, 0)),
                pl.BlockSpec((T, D_MODEL), lambda i, f, te_ref: (0, 0)),
                pl.BlockSpec((1, _FB, D_MODEL), lambda i, f, te_ref: (te_ref[i], f, 0)),
                pl.BlockSpec((1, D_MODEL, _FB), lambda i, f, te_ref: (te_ref[i], 0, f)),
            ],
            out_specs=pl.BlockSpec((TILE, D_MODEL), lambda i, f, te_ref: (i, 0)),
            scratch_shapes=[pltpu.VMEM((TILE, D_MODEL), jnp.float32)],
        ),
        out_shape=jax.ShapeDtypeStruct((ROWS, D_MODEL), jnp.float32),
    )(te, tok.reshape(NT, 1, TILE), x_flat, w1, w2)


_CCH = 32


def _combine_body(
    pos1_hbm, pos2_hbm, p1_hbm, p2_hbm, ys_hbm, out_hbm,
    pv1, pv2, g1_v, g2_v, a_v, b_v, o_v, sem,
):
    core = _axis("c")
    sub = _axis("s")
    wid = sub * 2 + core
    per = T // 32

    def step(j, _):
        base = wid * per + j * _CCH
        pltpu.sync_copy(pos1_hbm.at[_ds8(base, _CCH)], pv1)
        pltpu.sync_copy(pos2_hbm.at[_ds8(base, _CCH)], pv2)
        pltpu.sync_copy(p1_hbm.at[_ds8(base, _CCH)], g1_v)
        pltpu.sync_copy(p2_hbm.at[_ds8(base, _CCH)], g2_v)
        _gather_rows(ys_hbm, pv1, a_v, sem)
        _gather_rows(ys_hbm, pv2, b_v, sem)

        def row_step(r, _):
            lm = jnp.arange(L, dtype=jnp.int32) == (r % L)
            ch = _ds8((r // L) * L, L)
            g1 = jnp.full((L,), jnp.sum(jnp.where(lm, g1_v[ch], 0.0)), jnp.float32)
            g2 = jnp.full((L,), jnp.sum(jnp.where(lm, g2_v[ch], 0.0)), jnp.float32)

            def col_step(d, _):
                o_v[r, pl.ds(d * L, L)] = (
                    g1 * a_v[r, pl.ds(d * L, L)] + g2 * b_v[r, pl.ds(d * L, L)]
                )
                return 0

            lax.fori_loop(0, D_MODEL // L, col_step, 0)
            return 0

        lax.fori_loop(0, _CCH, row_step, 0)
        pltpu.sync_copy(o_v, out_hbm.at[_ds8(base, _CCH)])
        return 0

    lax.fori_loop(0, per // _CCH, step, 0)


def _combine(pos1, pos2, p1, p2, ys):
    f = pl.kernel(
        _combine_body,
        out_type=jax.ShapeDtypeStruct((T, D_MODEL), jnp.float32),
        mesh=_sc_mesh(),
        compiler_params=pltpu.CompilerParams(needs_layout_passes=False),
        scratch_types=[
            pltpu.VMEM((_CCH,), jnp.int32),
            pltpu.VMEM((_CCH,), jnp.int32),
            pltpu.VMEM((_CCH,), jnp.float32),
            pltpu.VMEM((_CCH,), jnp.float32),
            pltpu.VMEM((_CCH, D_MODEL), jnp.float32),
            pltpu.VMEM((_CCH, D_MODEL), jnp.float32),
            pltpu.VMEM((_CCH, D_MODEL), jnp.float32),
            pltpu.SemaphoreType.DMA,
        ],
    )
    return f(pos1, pos2, p1, p2, ys)


def kernel(x, router_w, w1, w2):
    B, S, D = x.shape
    x_flat = x.reshape(-1, D)
    i1, i2, p1, p2, aux = _router(x_flat, router_w)
    i1 = i1.reshape(T)
    i2 = i2.reshape(T)
    tok, te, pos1m, pos2m = _dispatch(i1, i2)
    ys = _ffn(te, tok, x_flat, w1, w2)
    out = _combine(pos1m.reshape(T), pos2m.reshape(T), p1.reshape(T), p2.reshape(T), ys)
    return out.reshape(B, S, D), aux[0, 0]
